# Initial kernel scaffold; baseline (speedup 1.0000x reference)
import jax
import jax.numpy as jnp
from jax.experimental import pallas as pl


def kernel(xi, values, mins, ranges):
    raise NotImplementedError("write your pallas kernel here")


# speedup vs baseline: 24.4112x; 24.4112x over previous
import functools

import jax
import jax.numpy as jnp
from jax import lax
from jax.experimental import pallas as pl
from jax.experimental.pallas import tpu as pltpu
from jax.experimental.pallas import tpu_sc as plsc

NC = 2
NS = 16
NW = NC * NS
CH = 2048
ROW = 128
ROWS = CH // ROW
SUB = ROW // 16


@functools.lru_cache(maxsize=None)
def _build(NQ: int, H: int, W: int):
    QPW = NQ // NW
    NCH = QPW // CH

    @functools.partial(
        pl.kernel,
        out_type=jax.ShapeDtypeStruct((NQ,), jnp.float32),
        mesh=plsc.VectorSubcoreMesh(core_axis_name="c", subcore_axis_name="s"),
        compiler_params=pltpu.CompilerParams(
            needs_layout_passes=False, use_tc_tiling_on_sc=False),
        scratch_types=[
            pltpu.VMEM((2 * CH,), jnp.float32),
            pltpu.VMEM((ROWS, ROW), jnp.int32),
            pltpu.VMEM((ROWS, ROW), jnp.int32),
            pltpu.VMEM((CH,), jnp.int32),
            pltpu.VMEM((CH,), jnp.float32),
            pltpu.VMEM((CH,), jnp.float32),
            pltpu.VMEM((CH, 8), jnp.float32),
            pltpu.VMEM((CH, 8), jnp.float32),
            pltpu.VMEM((CH,), jnp.float32),
            pltpu.VMEM((64,), jnp.float32),
            pltpu.SemaphoreType.DMA,
        ],
    )
    def body(xi_hbm, pairs_hbm, params_hbm, out_hbm,
             xi_v, i00_v, i10_v, col_v, fx_v, fy_v, p00_v, p10_v, out_v,
             params_v, sem):
        wid = lax.axis_index("s") * NC + lax.axis_index("c")
        pltpu.sync_copy(params_hbm, params_v)
        lane = lax.iota(jnp.int32, 16)
        mink = params_v[pl.ds(0, 16)]
        minz = params_v[pl.ds(16, 16)]
        rngk = params_v[pl.ds(32, 16)]
        rngz = params_v[pl.ds(48, 16)]

        def chunk_body(c, carry):
            qbase = wid * QPW + c * CH
            pltpu.sync_copy(xi_hbm.at[pl.ds(2 * qbase, 2 * CH)], xi_v)

            def row_body(j, carry2):
                for k in range(SUB):
                    off = j * ROW + k * 16
                    idx2 = (off + lane) * 2
                    xk = plsc.load_gather(xi_v, [idx2])
                    xz = plsc.load_gather(xi_v, [idx2 + 1])
                    gy = 2.0 * (xk - mink) / rngk - 1.0
                    gx = 2.0 * (xz - minz) / rngz - 1.0
                    ix = jnp.clip((gx + 1.0) * 0.5 * (W - 1), 0.0, W - 1.0)
                    iy = jnp.clip((gy + 1.0) * 0.5 * (H - 1), 0.0, H - 1.0)
                    x0 = ix.astype(jnp.int32)
                    y0 = iy.astype(jnp.int32)
                    fx = ix - x0.astype(jnp.float32)
                    fy = iy - y0.astype(jnp.float32)
                    y1 = jnp.minimum(y0 + 1, H - 1)
                    i00 = y0 * W + x0
                    i10 = y1 * W + x0
                    i00_v[j, pl.ds(k * 16, 16)] = i00 >> 2
                    i10_v[j, pl.ds(k * 16, 16)] = i10 >> 2
                    col_v[pl.ds(off, 16)] = (i00 & 3) * 2
                    fx_v[pl.ds(off, 16)] = fx
                    fy_v[pl.ds(off, 16)] = fy
                pltpu.make_async_copy(
                    pairs_hbm.at[i00_v.at[j]],
                    p00_v.at[pl.ds(j * ROW, ROW)], sem).start()
                pltpu.make_async_copy(
                    pairs_hbm.at[i10_v.at[j]],
                    p10_v.at[pl.ds(j * ROW, ROW)], sem).start()
                return carry2

            lax.fori_loop(0, ROWS, row_body, 0)

            def wait_body(j, carry2):
                pltpu.make_async_copy(
                    pairs_hbm.at[i00_v.at[j]],
                    p00_v.at[pl.ds(j * ROW, ROW)], sem).wait()
                pltpu.make_async_copy(
                    pairs_hbm.at[i10_v.at[j]],
                    p10_v.at[pl.ds(j * ROW, ROW)], sem).wait()
                return carry2

            lax.fori_loop(0, ROWS, wait_body, 0)

            def blend_body(j, carry2):
                for k in range(SUB):
                    off = j * ROW + k * 16
                    q16 = off + lane
                    cc = col_v[pl.ds(off, 16)]
                    v00 = plsc.load_gather(p00_v, [q16, cc])
                    v01 = plsc.load_gather(p00_v, [q16, cc + 1])
                    v10 = plsc.load_gather(p10_v, [q16, cc])
                    v11 = plsc.load_gather(p10_v, [q16, cc + 1])
                    fx = fx_v[pl.ds(off, 16)]
                    fy = fy_v[pl.ds(off, 16)]
                    top = v00 + fx * (v01 - v00)
                    bot = v10 + fx * (v11 - v10)
                    out_v[pl.ds(off, 16)] = top + fy * (bot - top)
                return carry2

            lax.fori_loop(0, ROWS, blend_body, 0)
            pltpu.sync_copy(out_v, out_hbm.at[pl.ds(qbase, CH)])
            return carry

        lax.fori_loop(0, NCH, chunk_body, 0)

    return body


def kernel(xi, values, mins, ranges):
    NQ = xi.shape[0]
    H, W = values.shape
    flat = values.reshape(-1)
    shifted = jnp.concatenate([flat[1:], flat[-1:]])
    pairs = jnp.stack([flat, shifted], axis=1).reshape(H * W // 4, 8)
    params = jnp.repeat(
        jnp.concatenate([
            mins.astype(jnp.float32).reshape(-1),
            ranges.astype(jnp.float32).reshape(-1),
        ]),
        16,
    )
    out_flat = _build(NQ, H, W)(xi.reshape(-1), pairs, params)
    return out_flat.reshape(NQ, 1)


# speedup vs baseline: 199.5994x; 8.1765x over previous
import functools

import jax
import jax.numpy as jnp
from jax import lax
from jax.experimental import pallas as pl
from jax.experimental.pallas import tpu as pltpu
from jax.experimental.pallas import tpu_sc as plsc

NC = 2
NS = 16
NW = NC * NS
CH = 2048
ROW = 128
ROWS = CH // ROW
SUB = ROW // 16


@functools.lru_cache(maxsize=None)
def _build(NQ: int, H: int, W: int):
    QPW = NQ // NW
    NCH = QPW // CH
    N8 = H * W // 8

    @functools.partial(
        pl.kernel,
        out_type=jax.ShapeDtypeStruct((NQ,), jnp.float32),
        mesh=plsc.VectorSubcoreMesh(core_axis_name="c", subcore_axis_name="s"),
        compiler_params=pltpu.CompilerParams(
            needs_layout_passes=False, use_tc_tiling_on_sc=False),
        scratch_types=[
            pltpu.VMEM((CH,), jnp.float32),
            pltpu.VMEM((CH,), jnp.float32),
            pltpu.VMEM((ROWS, ROW), jnp.int32),
            pltpu.VMEM((ROWS, ROW), jnp.int32),
            pltpu.VMEM((ROWS, ROW), jnp.int32),
            pltpu.VMEM((ROWS, ROW), jnp.int32),
            pltpu.VMEM((CH,), jnp.int32),
            pltpu.VMEM((CH,), jnp.float32),
            pltpu.VMEM((CH,), jnp.float32),
            pltpu.VMEM((CH, 8), jnp.float32),
            pltpu.VMEM((CH, 8), jnp.float32),
            pltpu.VMEM((CH, 8), jnp.float32),
            pltpu.VMEM((CH, 8), jnp.float32),
            pltpu.VMEM((CH,), jnp.float32),
            pltpu.VMEM((64,), jnp.float32),
            pltpu.SemaphoreType.DMA,
        ],
    )
    def body(xi_hbm, vals_hbm, params_hbm, out_hbm,
             xk_v, xz_v, i00_v, i00n_v, i10_v, i10n_v, col_v, fx_v, fy_v,
             p00_v, p00n_v, p10_v, p10n_v, out_v, params_v, sem):
        wid = lax.axis_index("s") * NC + lax.axis_index("c")
        pltpu.sync_copy(params_hbm, params_v)
        lane = lax.iota(jnp.int32, 16)
        mink = params_v[pl.ds(0, 16)]
        minz = params_v[pl.ds(16, 16)]
        rngk = params_v[pl.ds(32, 16)]
        rngz = params_v[pl.ds(48, 16)]

        def chunk_body(c, carry):
            qbase = wid * QPW + c * CH
            pltpu.sync_copy(xi_hbm.at[pl.ds(qbase, CH)], xk_v)
            pltpu.sync_copy(xi_hbm.at[pl.ds(NQ + qbase, CH)], xz_v)

            def row_body(j, carry2):
                for k in range(SUB):
                    off = j * ROW + k * 16
                    xk = xk_v[pl.ds(off, 16)]
                    xz = xz_v[pl.ds(off, 16)]
                    gy = 2.0 * (xk - mink) / rngk - 1.0
                    gx = 2.0 * (xz - minz) / rngz - 1.0
                    ix = jnp.clip((gx + 1.0) * 0.5 * (W - 1), 0.0, W - 1.0)
                    iy = jnp.clip((gy + 1.0) * 0.5 * (H - 1), 0.0, H - 1.0)
                    x0 = ix.astype(jnp.int32)
                    y0 = iy.astype(jnp.int32)
                    fx = ix - x0.astype(jnp.float32)
                    fy = iy - y0.astype(jnp.float32)
                    y1 = jnp.minimum(y0 + 1, H - 1)
                    i00 = y0 * W + x0
                    i10 = y1 * W + x0
                    r00 = i00 >> 3
                    r10 = i10 >> 3
                    i00_v[j, pl.ds(k * 16, 16)] = r00
                    i00n_v[j, pl.ds(k * 16, 16)] = jnp.minimum(r00 + 1, N8 - 1)
                    i10_v[j, pl.ds(k * 16, 16)] = r10
                    i10n_v[j, pl.ds(k * 16, 16)] = jnp.minimum(r10 + 1, N8 - 1)
                    col_v[pl.ds(off, 16)] = i00 & 7
                    fx_v[pl.ds(off, 16)] = fx
                    fy_v[pl.ds(off, 16)] = fy
                dst = pl.ds(j * ROW, ROW)
                pltpu.make_async_copy(
                    vals_hbm.at[i00_v.at[j]], p00_v.at[dst], sem).start()
                pltpu.make_async_copy(
                    vals_hbm.at[i00n_v.at[j]], p00n_v.at[dst], sem).start()
                pltpu.make_async_copy(
                    vals_hbm.at[i10_v.at[j]], p10_v.at[dst], sem).start()
                pltpu.make_async_copy(
                    vals_hbm.at[i10n_v.at[j]], p10n_v.at[dst], sem).start()
                return carry2

            lax.fori_loop(0, ROWS, row_body, 0)

            def wait_body(j, carry2):
                dst = pl.ds(j * ROW, ROW)
                pltpu.make_async_copy(
                    vals_hbm.at[i00_v.at[j]], p00_v.at[dst], sem).wait()
                pltpu.make_async_copy(
                    vals_hbm.at[i00n_v.at[j]], p00n_v.at[dst], sem).wait()
                pltpu.make_async_copy(
                    vals_hbm.at[i10_v.at[j]], p10_v.at[dst], sem).wait()
                pltpu.make_async_copy(
                    vals_hbm.at[i10n_v.at[j]], p10n_v.at[dst], sem).wait()
                return carry2

            lax.fori_loop(0, ROWS, wait_body, 0)

            def blend_body(j, carry2):
                for k in range(SUB):
                    off = j * ROW + k * 16
                    q16 = off + lane
                    cc = col_v[pl.ds(off, 16)]
                    cross = cc == 7
                    cn = (cc + 1) & 7
                    v00 = plsc.load_gather(p00_v, [q16, cc])
                    v01a = plsc.load_gather(p00_v, [q16, cn])
                    v01b = plsc.load_gather(p00n_v, [q16, cn])
                    v10 = plsc.load_gather(p10_v, [q16, cc])
                    v11a = plsc.load_gather(p10_v, [q16, cn])
                    v11b = plsc.load_gather(p10n_v, [q16, cn])
                    v01 = jnp.where(cross, v01b, v01a)
                    v11 = jnp.where(cross, v11b, v11a)
                    fx = fx_v[pl.ds(off, 16)]
                    fy = fy_v[pl.ds(off, 16)]
                    top = v00 + fx * (v01 - v00)
                    bot = v10 + fx * (v11 - v10)
                    out_v[pl.ds(off, 16)] = top + fy * (bot - top)
                return carry2

            lax.fori_loop(0, ROWS, blend_body, 0)
            pltpu.sync_copy(out_v, out_hbm.at[pl.ds(qbase, CH)])
            return carry

        lax.fori_loop(0, NCH, chunk_body, 0)

    return body


def kernel(xi, values, mins, ranges):
    NQ = xi.shape[0]
    H, W = values.shape
    xi_planes = xi.T.reshape(-1)
    vals8 = values.reshape(H * W // 8, 8)
    params = jnp.repeat(
        jnp.concatenate([
            mins.astype(jnp.float32).reshape(-1),
            ranges.astype(jnp.float32).reshape(-1),
        ]),
        16,
    )
    out_flat = _build(NQ, H, W)(xi_planes, vals8, params)
    return out_flat.reshape(NQ, 1)


# speedup vs baseline: 199.9952x; 1.0020x over previous
import functools

import jax
import jax.numpy as jnp
from jax import lax
from jax.experimental import pallas as pl
from jax.experimental.pallas import tpu as pltpu
from jax.experimental.pallas import tpu_sc as plsc

NC = 2
NS = 16
NW = NC * NS
CH = 2048
ROW = 128
ROWS = CH // ROW
SUB = ROW // 16


@functools.lru_cache(maxsize=None)
def _build(NQ: int, H: int, W: int):
    QPW = NQ // NW
    NCH = QPW // CH
    N8 = H * W // 8

    @functools.partial(
        pl.kernel,
        out_type=jax.ShapeDtypeStruct((NQ,), jnp.float32),
        mesh=plsc.VectorSubcoreMesh(core_axis_name="c", subcore_axis_name="s"),
        compiler_params=pltpu.CompilerParams(
            needs_layout_passes=False, use_tc_tiling_on_sc=False),
        scratch_types=[
            pltpu.VMEM((CH,), jnp.float32),
            pltpu.VMEM((CH,), jnp.float32),
            pltpu.VMEM((ROWS, ROW), jnp.int32),
            pltpu.VMEM((ROWS, ROW), jnp.int32),
            pltpu.VMEM((ROWS, ROW), jnp.int32),
            pltpu.VMEM((ROWS, ROW), jnp.int32),
            pltpu.VMEM((CH,), jnp.int32),
            pltpu.VMEM((CH,), jnp.float32),
            pltpu.VMEM((CH,), jnp.float32),
            pltpu.VMEM((CH, 8), jnp.float32),
            pltpu.VMEM((CH, 8), jnp.float32),
            pltpu.VMEM((CH, 8), jnp.float32),
            pltpu.VMEM((CH, 8), jnp.float32),
            pltpu.VMEM((CH,), jnp.float32),
            pltpu.VMEM((64,), jnp.float32),
            pltpu.SemaphoreType.DMA,
        ],
    )
    def body(xi_hbm, vals_hbm, params_hbm, out_hbm,
             xk_v, xz_v, i00_v, i00n_v, i10_v, i10n_v, col_v, fx_v, fy_v,
             p00_v, p00n_v, p10_v, p10n_v, out_v, params_v, sem):
        wid = lax.axis_index("s") * NC + lax.axis_index("c")
        pltpu.sync_copy(params_hbm, params_v)
        lane = lax.iota(jnp.int32, 16)
        ay = params_v[pl.ds(0, 16)]
        by = params_v[pl.ds(16, 16)]
        az = params_v[pl.ds(32, 16)]
        bz = params_v[pl.ds(48, 16)]

        def chunk_body(c, carry):
            qbase = wid * QPW + c * CH
            pltpu.sync_copy(xi_hbm.at[pl.ds(qbase, CH)], xk_v)
            pltpu.sync_copy(xi_hbm.at[pl.ds(NQ + qbase, CH)], xz_v)

            def row_body(j, carry2):
                for k in range(SUB):
                    off = j * ROW + k * 16
                    xk = xk_v[pl.ds(off, 16)]
                    xz = xz_v[pl.ds(off, 16)]
                    ix = jnp.clip(xz * az + bz, 0.0, W - 1.0)
                    iy = jnp.clip(xk * ay + by, 0.0, H - 1.0)
                    x0 = ix.astype(jnp.int32)
                    y0 = iy.astype(jnp.int32)
                    fx = ix - x0.astype(jnp.float32)
                    fy = iy - y0.astype(jnp.float32)
                    y1 = jnp.minimum(y0 + 1, H - 1)
                    i00 = y0 * W + x0
                    i10 = y1 * W + x0
                    r00 = i00 >> 3
                    r10 = i10 >> 3
                    i00_v[j, pl.ds(k * 16, 16)] = r00
                    i00n_v[j, pl.ds(k * 16, 16)] = jnp.minimum(r00 + 1, N8 - 1)
                    i10_v[j, pl.ds(k * 16, 16)] = r10
                    i10n_v[j, pl.ds(k * 16, 16)] = jnp.minimum(r10 + 1, N8 - 1)
                    col_v[pl.ds(off, 16)] = i00 & 7
                    fx_v[pl.ds(off, 16)] = fx
                    fy_v[pl.ds(off, 16)] = fy
                dst = pl.ds(j * ROW, ROW)
                pltpu.make_async_copy(
                    vals_hbm.at[i00_v.at[j]], p00_v.at[dst], sem).start()
                pltpu.make_async_copy(
                    vals_hbm.at[i00n_v.at[j]], p00n_v.at[dst], sem).start()
                pltpu.make_async_copy(
                    vals_hbm.at[i10_v.at[j]], p10_v.at[dst], sem).start()
                pltpu.make_async_copy(
                    vals_hbm.at[i10n_v.at[j]], p10n_v.at[dst], sem).start()
                return carry2

            lax.fori_loop(0, ROWS, row_body, 0)

            def wait_body(j, carry2):
                dst = pl.ds(j * ROW, ROW)
                pltpu.make_async_copy(
                    vals_hbm.at[i00_v.at[j]], p00_v.at[dst], sem).wait()
                pltpu.make_async_copy(
                    vals_hbm.at[i00n_v.at[j]], p00n_v.at[dst], sem).wait()
                pltpu.make_async_copy(
                    vals_hbm.at[i10_v.at[j]], p10_v.at[dst], sem).wait()
                pltpu.make_async_copy(
                    vals_hbm.at[i10n_v.at[j]], p10n_v.at[dst], sem).wait()
                return carry2

            lax.fori_loop(0, ROWS, wait_body, 0)

            def blend_body(j, carry2):
                for k in range(SUB):
                    off = j * ROW + k * 16
                    q16 = off + lane
                    cc = col_v[pl.ds(off, 16)]
                    cross = cc == 7
                    cn = (cc + 1) & 7
                    v00 = plsc.load_gather(p00_v, [q16, cc])
                    v01a = plsc.load_gather(p00_v, [q16, cn])
                    v01b = plsc.load_gather(p00n_v, [q16, cn])
                    v10 = plsc.load_gather(p10_v, [q16, cc])
                    v11a = plsc.load_gather(p10_v, [q16, cn])
                    v11b = plsc.load_gather(p10n_v, [q16, cn])
                    v01 = jnp.where(cross, v01b, v01a)
                    v11 = jnp.where(cross, v11b, v11a)
                    fx = fx_v[pl.ds(off, 16)]
                    fy = fy_v[pl.ds(off, 16)]
                    top = v00 + fx * (v01 - v00)
                    bot = v10 + fx * (v11 - v10)
                    out_v[pl.ds(off, 16)] = top + fy * (bot - top)
                return carry2

            lax.fori_loop(0, ROWS, blend_body, 0)
            pltpu.sync_copy(out_v, out_hbm.at[pl.ds(qbase, CH)])
            return carry

        lax.fori_loop(0, NCH, chunk_body, 0)

    return body


def kernel(xi, values, mins, ranges):
    NQ = xi.shape[0]
    H, W = values.shape
    xi_planes = xi.T.reshape(-1)
    vals8 = values.reshape(H * W // 8, 8)
    m = mins.astype(jnp.float32).reshape(-1)
    r = ranges.astype(jnp.float32).reshape(-1)
    ay = (H - 1.0) / r[0]
    az = (W - 1.0) / r[1]
    params = jnp.repeat(
        jnp.stack([ay, -m[0] * ay, az, -m[1] * az]),
        16,
    )
    out_flat = _build(NQ, H, W)(xi_planes, vals8, params)
    return out_flat.reshape(NQ, 1)
